# Initial kernel scaffold; baseline (speedup 1.0000x reference)
import jax
import jax.numpy as jnp
from jax.experimental import pallas as pl


def kernel(x, embedding_table):
    raise NotImplementedError("write your pallas kernel here")


# speedup vs baseline: 1.4838x; 1.4838x over previous
import functools

import numpy as np
import jax
import jax.numpy as jnp
from jax import lax
from jax.experimental import pallas as pl
from jax.experimental.pallas import tpu as pltpu
from jax.experimental.pallas import tpu_sc as plsc

_VOCAB = 100000
_D = 128
_SEQ = 200
_BATCH = 1024
_NW = 32
_CHUNK = 100
_TOK = _BATCH * _SEQ
_TPW = _TOK // _NW
_NCH = _TPW // _CHUNK
_SCALE = float(np.sqrt(float(_D)))


def _pos_table() -> np.ndarray:
    d = np.arange(_D)
    even = (d % 2 == 0).astype(np.float64)
    odd = (d % 2 == 1).astype(np.float64)
    rate = 1.0 / (10000.0 ** (d[np.newaxis, :] / _D))
    rads = np.arange(_SEQ)[:, np.newaxis] * rate
    return (np.sin(rads) * even + np.cos(rads) * odd).astype(np.float32)


_POS = _pos_table()

_mesh = plsc.VectorSubcoreMesh(core_axis_name="c", subcore_axis_name="s")


@functools.partial(
    pl.kernel,
    mesh=_mesh,
    out_type=jax.ShapeDtypeStruct((_NW, _NCH, _CHUNK, _D), jnp.float32),
    scratch_types=[
        pltpu.VMEM((_NCH, _CHUNK), jnp.int32),
        pltpu.VMEM((_SEQ, _D), jnp.float32),
        pltpu.VMEM((_CHUNK, _D), jnp.float32),
        pltpu.SemaphoreType.DMA,
    ],
)
def _emb_lookup(idx_hbm, tab_hbm, pos_hbm, out_hbm, idx_v, pos_v, rows_v, sem):
    wid = lax.axis_index("s") * 2 + lax.axis_index("c")
    pltpu.sync_copy(pos_hbm, pos_v)
    pltpu.sync_copy(idx_hbm.at[wid], idx_v)

    def chunk_body(g, carry):
        pltpu.async_copy(tab_hbm.at[idx_v.at[g]], rows_v, sem).wait()
        po = (g % 2) * _CHUNK

        def row_body(r, c):
            for j in range(_D // 16):
                sl = pl.ds(j * 16, 16)
                rows_v[r, sl] = rows_v[r, sl] * _SCALE + pos_v[po + r, sl]
            return c

        lax.fori_loop(0, _CHUNK, row_body, 0)
        pltpu.sync_copy(rows_v, out_hbm.at[wid, g])
        return carry

    lax.fori_loop(0, _NCH, chunk_body, 0)


def kernel(x, embedding_table):
    idx = x.reshape(_NW, _NCH, _CHUNK).astype(jnp.int32)
    pos = jnp.asarray(_POS)
    out = _emb_lookup(idx, embedding_table, pos)
    return out.reshape(_BATCH, _SEQ, _D)


# speedup vs baseline: 3.8054x; 2.5647x over previous
import functools

import numpy as np
import jax
import jax.numpy as jnp
from jax import lax
from jax.experimental import pallas as pl
from jax.experimental.pallas import tpu as pltpu
from jax.experimental.pallas import tpu_sc as plsc

_VOCAB = 100000
_D = 128
_SEQ = 200
_BATCH = 1024
_NW = 32
_CHUNK = 100
_TOK = _BATCH * _SEQ
_TPW = _TOK // _NW
_NCH = _TPW // _CHUNK
_NBUF = 4
_SCALE = float(np.sqrt(float(_D)))


def _pos_table() -> np.ndarray:
    d = np.arange(_D)
    even = (d % 2 == 0).astype(np.float64)
    odd = (d % 2 == 1).astype(np.float64)
    rate = 1.0 / (10000.0 ** (d[np.newaxis, :] / _D))
    rads = np.arange(_SEQ)[:, np.newaxis] * rate
    return (np.sin(rads) * even + np.cos(rads) * odd).astype(np.float32)


_POS = _pos_table()

_mesh = plsc.VectorSubcoreMesh(core_axis_name="c", subcore_axis_name="s")


@functools.partial(
    pl.kernel,
    mesh=_mesh,
    out_type=jax.ShapeDtypeStruct((_NW, _NCH, _CHUNK, _D), jnp.float32),
    scratch_types=[
        pltpu.VMEM((_NCH, _CHUNK), jnp.int32),
        pltpu.VMEM((_SEQ, _D), jnp.float32),
        pltpu.VMEM((_NBUF, _CHUNK, _D), jnp.float32),
        pltpu.SemaphoreType.DMA,
        pltpu.SemaphoreType.DMA,
        pltpu.SemaphoreType.DMA,
        pltpu.SemaphoreType.DMA,
        pltpu.SemaphoreType.DMA,
        pltpu.SemaphoreType.DMA,
        pltpu.SemaphoreType.DMA,
        pltpu.SemaphoreType.DMA,
    ],
)
def _emb_lookup(idx_hbm, tab_hbm, pos_hbm, out_hbm, idx_v, pos_v, buf,
                gs0, gs1, gs2, gs3, ss0, ss1, ss2, ss3):
    gsems = (gs0, gs1, gs2, gs3)
    ssems = (ss0, ss1, ss2, ss3)
    wid = lax.axis_index("s") * 2 + lax.axis_index("c")
    pltpu.sync_copy(pos_hbm, pos_v)
    pltpu.sync_copy(idx_hbm.at[wid], idx_v)

    def issue_gather(g, s):
        pltpu.async_copy(tab_hbm.at[idx_v.at[g]], buf.at[s], gsems[s])

    def wait_gather(s):
        pltpu.make_async_copy(out_hbm.at[0, 0], buf.at[s], gsems[s]).wait()

    def issue_store(g, s):
        pltpu.async_copy(buf.at[s], out_hbm.at[wid, g], ssems[s])

    def wait_store(s):
        pltpu.make_async_copy(buf.at[s], out_hbm.at[wid, 0], ssems[s]).wait()

    def compute(s, po):
        def row_body(r, c):
            for j in range(_D // 16):
                sl = pl.ds(j * 16, 16)
                buf[s, r, sl] = buf[s, r, sl] * _SCALE + pos_v[po + r, sl]
            return c

        lax.fori_loop(0, _CHUNK, row_body, 0)

    for b in range(_NBUF - 1):
        issue_gather(b, b)

    wait_gather(0)
    compute(0, 0)
    issue_gather(_NBUF - 1, _NBUF - 1)
    issue_store(0, 0)
    for b in range(1, _NBUF):
        wait_gather(b)
        compute(b, (b % 2) * _CHUNK)
        wait_store(b - 1)
        issue_gather(b + _NBUF - 1, b - 1)
        issue_store(b, b)

    def block_body(it, carry):
        g0 = it * _NBUF
        for b in range(_NBUF):
            g = g0 + b
            wait_gather(b)
            compute(b, (b % 2) * _CHUNK)
            wait_store((b - 1) % _NBUF)
            issue_gather(g + _NBUF - 1, (b - 1) % _NBUF)
            issue_store(g, b)
        return carry

    lax.fori_loop(1, _NCH // _NBUF - 1, block_body, 0)

    g0 = _NCH - _NBUF
    wait_gather(0)
    compute(0, 0)
    wait_store(_NBUF - 1)
    issue_gather(g0 + _NBUF - 1, _NBUF - 1)
    issue_store(g0, 0)
    for b in range(1, _NBUF):
        wait_gather(b)
        compute(b, (b % 2) * _CHUNK)
        wait_store(b - 1)
        issue_store(g0 + b, b)
    wait_store(_NBUF - 1)


def kernel(x, embedding_table):
    idx = x.reshape(_NW, _NCH, _CHUNK).astype(jnp.int32)
    pos = jnp.asarray(_POS)
    out = _emb_lookup(idx, embedding_table, pos)
    return out.reshape(_BATCH, _SEQ, _D)
